# Initial kernel scaffold; baseline (speedup 1.0000x reference)
import jax
import jax.numpy as jnp
from jax.experimental import pallas as pl


def kernel(x, edge_index, Winit, binit, Wc0, bc0, Wl0, bl0, Wc1, bc1, Wl1, bl1, Wf, bf):
    raise NotImplementedError("write your pallas kernel here")


# speedup vs baseline: 19.8449x; 19.8449x over previous
import functools

import jax
import jax.numpy as jnp
from jax import lax
from jax.experimental import pallas as pl
from jax.experimental.pallas import tpu as pltpu
from jax.experimental.pallas import tpu_sc as plsc

N = 10000
E = 320000
D = 128

NC = 2
NS = 16
L = 16
NW = NC * NS

NPAD = 10240
DUMMY = N
EPW = E // NW
CHUNK = 80
NCHUNK = EPW // CHUNK
HROW = NPAD // L
ZROWS = NPAD // NS
OROWS = N // NS

_MESH = plsc.VectorSubcoreMesh(
    core_axis_name="c", subcore_axis_name="s", num_cores=NC, num_subcores=NS)
_SC_PARAMS = pltpu.CompilerParams(needs_layout_passes=False)


HR = NPAD // D


@functools.partial(
    pl.kernel,
    out_type=jax.ShapeDtypeStruct((NC, HR, D), jnp.float32),
    mesh=_MESH,
    compiler_params=_SC_PARAMS,
    scratch_types=[
        pltpu.VMEM((NCHUNK, CHUNK), jnp.int32),
        pltpu.VMEM((NCHUNK, CHUNK), jnp.int32),
        pltpu.VMEM((HR, D), jnp.float32),
        pltpu.VMEM((8, CHUNK), jnp.int32),
        pltpu.VMEM_SHARED((HR, D), jnp.float32),
    ],
)
def _deg_kernel(ei_ref, out_ref, rows2d, cols2d, hist, idxs, shared):
    c = lax.axis_index("c")
    s = lax.axis_index("s")
    wid = c * NS + s

    pltpu.sync_copy(ei_ref.at[0, wid], rows2d)
    pltpu.sync_copy(ei_ref.at[1, wid], cols2d)

    zeros16 = jnp.zeros((L,), jnp.float32)

    def _zero_hist(i, carry):
        for m in range(D // L):
            hist[i, pl.ds(m * L, L)] = zeros16
        return carry

    lax.fori_loop(0, HR, _zero_hist, 0)

    lane = jnp.arange(L, dtype=jnp.int32)
    for k in range(8):
        for j in range(CHUNK // L):
            idxs[k, pl.ds(j * L, L)] = lane + L * j

    @pl.when(s == 0)
    def _():
        pltpu.sync_copy(hist, shared)
    plsc.subcore_barrier()

    ones16 = jnp.ones((L,), jnp.float32)

    def _accum(j, carry):
        for m in range(CHUNK // L):
            r = rows2d[j, pl.ds(m * L, L)]
            cc = cols2d[j, pl.ds(m * L, L)]
            keep = r != cc
            plsc.addupdate_scatter(
                hist, [lax.shift_right_logical(r, 7), lax.bitwise_and(r, 127)],
                ones16, mask=keep)
        return carry

    lax.fori_loop(0, NCHUNK, _accum, 0)

    pltpu.sync_copy(hist, shared.at[idxs.at[0]], add=True)
    plsc.subcore_barrier()

    @pl.when(s == 0)
    def _():
        pltpu.sync_copy(shared, out_ref.at[c])


@functools.partial(
    pl.kernel,
    out_type=jax.ShapeDtypeStruct((NC, N, D), jnp.float32),
    mesh=_MESH,
    compiler_params=_SC_PARAMS,
    scratch_types=[
        pltpu.VMEM((NCHUNK, CHUNK), jnp.int32),
        pltpu.VMEM((NCHUNK, CHUNK), jnp.int32),
        pltpu.VMEM((CHUNK, D), jnp.float32),
        pltpu.VMEM_SHARED((NPAD, D), jnp.float32),
        pltpu.SemaphoreType.DMA,
    ],
)
def _spmm_kernel(sp_ref, ei_ref, out_ref, rows2d, cols2d, gbuf0, acc, sem0):
    c = lax.axis_index("c")
    s = lax.axis_index("s")
    wid = c * NS + s

    pltpu.sync_copy(ei_ref.at[0, wid], rows2d)
    pltpu.sync_copy(ei_ref.at[1, wid], cols2d)

    def _fix(j, carry):
        for m in range(CHUNK // L):
            r = rows2d[j, pl.ds(m * L, L)]
            cc = cols2d[j, pl.ds(m * L, L)]
            rows2d[j, pl.ds(m * L, L)] = jnp.where(r == cc, DUMMY, r)
        return carry

    lax.fori_loop(0, NCHUNK, _fix, 0)

    zeros16 = jnp.zeros((L,), jnp.float32)

    def _zero_gbuf(i, carry):
        for m in range(D // L):
            gbuf0[i, pl.ds(m * L, L)] = zeros16
        return carry

    lax.fori_loop(0, CHUNK, _zero_gbuf, 0)

    def _zero_acc(k, carry):
        pltpu.sync_copy(gbuf0, acc.at[pl.ds(s * ZROWS + k * CHUNK, CHUNK)])
        return carry

    lax.fori_loop(0, ZROWS // CHUNK, _zero_acc, 0)
    plsc.subcore_barrier()

    def _step(j, carry):
        pltpu.async_copy(sp_ref.at[cols2d.at[j]], gbuf0, sem0)
        pltpu.make_async_copy(sp_ref.at[cols2d.at[j]], gbuf0, sem0).wait()
        pltpu.sync_copy(gbuf0, acc.at[rows2d.at[j]], add=True)
        return carry

    lax.fori_loop(0, NCHUNK, _step, 0)
    plsc.subcore_barrier()

    base = pl.multiple_of(s * ZROWS, 8)

    @pl.when(s < NS - 1)
    def _():
        pltpu.sync_copy(acc.at[pl.ds(base, ZROWS)],
                        out_ref.at[c, pl.ds(base, ZROWS)])

    @pl.when(s == NS - 1)
    def _():
        pltpu.sync_copy(acc.at[pl.ds(ZROWS * (NS - 1), N - ZROWS * (NS - 1))],
                        out_ref.at[c, pl.ds(ZROWS * (NS - 1),
                                            N - ZROWS * (NS - 1))])


BM = 1000
GRID = N // BM

_row_spec = pl.BlockSpec((BM, D), lambda i: (i, 0))
_col_spec = pl.BlockSpec((BM, 1), lambda i: (i, 0))
_w_spec = pl.BlockSpec((D, D), lambda i: (0, 0))
_b_spec = pl.BlockSpec((1, D), lambda i: (0, 0))
_pair_spec = pl.BlockSpec((2, BM, D), lambda i: (0, i, 0))
_dpair_spec = pl.BlockSpec((2, BM, 1), lambda i: (0, i, 0))


def _k0_body(x_ref, dg_ref, wi_ref, bi_ref, wc_ref, h_ref, sp_ref, dis_ref):
    deg = dg_ref[0] + dg_ref[1] + 1.0
    dis = lax.rsqrt(deg)
    h = jnp.dot(x_ref[...], wi_ref[...],
                preferred_element_type=jnp.float32) + bi_ref[...]
    h_ref[...] = h
    dis_ref[...] = dis
    sp_ref[...] = dis * jnp.dot(h, wc_ref[...],
                                preferred_element_type=jnp.float32)


_k0 = pl.pallas_call(
    _k0_body,
    grid=(GRID,),
    in_specs=[_row_spec, _dpair_spec, _w_spec, _b_spec, _w_spec],
    out_specs=[_row_spec, _row_spec, _col_spec],
    out_shape=[
        jax.ShapeDtypeStruct((N, D), jnp.float32),
        jax.ShapeDtypeStruct((N, D), jnp.float32),
        jax.ShapeDtypeStruct((N, 1), jnp.float32),
    ],
)


def _k1_body(p_ref, sp_ref, h_ref, dis_ref, wl_ref, bl_ref, bc_ref, wc1_ref,
             h1_ref, s1p_ref):
    dis = dis_ref[...]
    t = (dis * (p_ref[0] + p_ref[1] + sp_ref[...]) + bc_ref[...]
         + jnp.dot(h_ref[...], wl_ref[...], preferred_element_type=jnp.float32)
         + bl_ref[...])
    h1 = jnp.maximum(t, 0.0) + h_ref[...]
    h1_ref[...] = h1
    s1p_ref[...] = dis * jnp.dot(h1, wc1_ref[...],
                                 preferred_element_type=jnp.float32)


_k1 = pl.pallas_call(
    _k1_body,
    grid=(GRID,),
    in_specs=[_pair_spec, _row_spec, _row_spec, _col_spec, _w_spec, _b_spec,
              _b_spec, _w_spec],
    out_specs=[_row_spec, _row_spec],
    out_shape=[
        jax.ShapeDtypeStruct((N, D), jnp.float32),
        jax.ShapeDtypeStruct((N, D), jnp.float32),
    ],
)


def _k2_body(p_ref, sp_ref, h_ref, dis_ref, wl_ref, bl_ref, bc_ref, wf_ref,
             bf_ref, out_ref):
    dis = dis_ref[...]
    t = (dis * (p_ref[0] + p_ref[1] + sp_ref[...]) + bc_ref[...]
         + jnp.dot(h_ref[...], wl_ref[...], preferred_element_type=jnp.float32)
         + bl_ref[...])
    h2 = jnp.maximum(t, 0.0) + h_ref[...]
    out_ref[...] = jnp.dot(h2, wf_ref[...],
                           preferred_element_type=jnp.float32) + bf_ref[...]


_k2 = pl.pallas_call(
    _k2_body,
    grid=(GRID,),
    in_specs=[_pair_spec, _row_spec, _row_spec, _col_spec, _w_spec, _b_spec,
              _b_spec, _w_spec, _b_spec],
    out_specs=pl.BlockSpec((BM, D), lambda i: (i, 0)),
    out_shape=jax.ShapeDtypeStruct((N, D), jnp.float32),
)


def _spmm_jnp(sp, edge_index):
    row, col = edge_index[0], edge_index[1]
    keep = row != col
    acc = jnp.zeros_like(sp).at[row].add(
        jnp.where(keep[:, None], sp[col], 0.0))
    return jnp.stack([acc, jnp.zeros_like(acc)])


def kernel(x, edge_index, Winit, binit, Wc0, bc0, Wl0, bl0, Wc1, bc1, Wl1,
           bl1, Wf, bf):
    ei = edge_index.reshape(2, NW, NCHUNK, CHUNK)

    degp = _deg_kernel(ei)
    deg3 = degp.reshape(NC, NPAD, 1)[:, :N]

    h, s0p, dis = _k0(x, deg3, Winit, binit.reshape(1, D), Wc0)
    parts0 = _spmm_kernel(s0p, ei)
    h1, s1p = _k1(parts0, s0p, h, dis, Wl0, bl0.reshape(1, D),
                  bc0.reshape(1, D), Wc1)
    parts1 = _spmm_kernel(s1p, ei)
    out = _k2(parts1, s1p, h1, dis, Wl1, bl1.reshape(1, D),
              bc1.reshape(1, D), Wf, bf.reshape(1, D))
    return out


# speedup vs baseline: 28.7670x; 1.4496x over previous
import functools

import jax
import jax.numpy as jnp
from jax import lax
from jax.experimental import pallas as pl
from jax.experimental.pallas import tpu as pltpu
from jax.experimental.pallas import tpu_sc as plsc

N = 10000
E = 320000
D = 128

NC = 2
NS = 16
L = 16
NW = NC * NS

NPAD = 10240
DUMMY = N
EPW = E // NW
CHUNK = 80
NCHUNK = EPW // CHUNK
BCHUNK = 25
NBLK = NCHUNK // BCHUNK
HROW = NPAD // L
ZROWS = NPAD // NS
OROWS = N // NS

_MESH = plsc.VectorSubcoreMesh(
    core_axis_name="c", subcore_axis_name="s", num_cores=NC, num_subcores=NS)
_SC_PARAMS = pltpu.CompilerParams(needs_layout_passes=False)


HR = NPAD // D


@functools.partial(
    pl.kernel,
    out_type=jax.ShapeDtypeStruct((NC, HR, D), jnp.float32),
    mesh=_MESH,
    compiler_params=_SC_PARAMS,
    scratch_types=[
        pltpu.VMEM((NCHUNK, CHUNK), jnp.int32),
        pltpu.VMEM((NCHUNK, CHUNK), jnp.int32),
        pltpu.VMEM((HR, D), jnp.float32),
        pltpu.VMEM((8, CHUNK), jnp.int32),
        pltpu.VMEM_SHARED((HR, D), jnp.float32),
    ],
)
def _deg_kernel(ei_ref, out_ref, rows2d, cols2d, hist, idxs, shared):
    c = lax.axis_index("c")
    s = lax.axis_index("s")
    wid = c * NS + s

    pltpu.sync_copy(ei_ref.at[0, wid], rows2d)
    pltpu.sync_copy(ei_ref.at[1, wid], cols2d)

    zeros16 = jnp.zeros((L,), jnp.float32)

    def _zero_hist(i, carry):
        for m in range(D // L):
            hist[i, pl.ds(m * L, L)] = zeros16
        return carry

    lax.fori_loop(0, HR, _zero_hist, 0)

    lane = jnp.arange(L, dtype=jnp.int32)
    for k in range(8):
        for j in range(CHUNK // L):
            idxs[k, pl.ds(j * L, L)] = lane + L * j

    @pl.when(s == 0)
    def _():
        pltpu.sync_copy(hist, shared)
    plsc.subcore_barrier()

    ones16 = jnp.ones((L,), jnp.float32)

    def _accum(j, carry):
        for m in range(CHUNK // L):
            r = rows2d[j, pl.ds(m * L, L)]
            cc = cols2d[j, pl.ds(m * L, L)]
            keep = r != cc
            plsc.addupdate_scatter(
                hist, [lax.shift_right_logical(r, 7), lax.bitwise_and(r, 127)],
                ones16, mask=keep)
        return carry

    lax.fori_loop(0, NCHUNK, _accum, 0)

    pltpu.sync_copy(hist, shared.at[idxs.at[0]], add=True)
    plsc.subcore_barrier()

    @pl.when(s == 0)
    def _():
        pltpu.sync_copy(shared, out_ref.at[c])


@functools.partial(
    pl.kernel,
    out_type=jax.ShapeDtypeStruct((NC, N, D), jnp.float32),
    mesh=_MESH,
    compiler_params=_SC_PARAMS,
    scratch_types=[
        pltpu.VMEM((BCHUNK, CHUNK), jnp.int32),
        pltpu.VMEM((BCHUNK, CHUNK), jnp.int32),
        pltpu.VMEM((CHUNK, D), jnp.float32),
        pltpu.VMEM((CHUNK, D), jnp.float32),
        pltpu.VMEM_SHARED((NPAD, D), jnp.float32),
        pltpu.SemaphoreType.DMA,
        pltpu.SemaphoreType.DMA,
    ],
)
def _spmm_kernel(sp_ref, ei_ref, out_ref, rowsb, colsb, gbufA, gbufB, acc,
                 semA, semB):
    c = lax.axis_index("c")
    s = lax.axis_index("s")
    wid = c * NS + s

    zeros16 = jnp.zeros((L,), jnp.float32)

    def _zero_gbuf(i, carry):
        for m in range(D // L):
            gbufA[i, pl.ds(m * L, L)] = zeros16
        return carry

    lax.fori_loop(0, CHUNK, _zero_gbuf, 0)

    def _zero_acc(k, carry):
        pltpu.sync_copy(gbufA, acc.at[pl.ds(s * ZROWS + k * CHUNK, CHUNK)])
        return carry

    lax.fori_loop(0, ZROWS // CHUNK, _zero_acc, 0)
    plsc.subcore_barrier()

    for b in range(NBLK):
        pltpu.sync_copy(ei_ref.at[0, wid, b], rowsb)
        pltpu.sync_copy(ei_ref.at[1, wid, b], colsb)

        def _fix(j, carry):
            for m in range(CHUNK // L):
                r = rowsb[j, pl.ds(m * L, L)]
                cc = colsb[j, pl.ds(m * L, L)]
                rowsb[j, pl.ds(m * L, L)] = jnp.where(r == cc, DUMMY, r)
            return carry

        lax.fori_loop(0, BCHUNK, _fix, 0)

        pltpu.async_copy(sp_ref.at[colsb.at[0]], gbufA, semA)

        def _step(t, carry):
            j0 = 2 * t
            j1 = 2 * t + 1

            @pl.when(j1 < BCHUNK)
            def _():
                pltpu.async_copy(sp_ref.at[colsb.at[j1]], gbufB, semB)

            pltpu.make_async_copy(sp_ref.at[colsb.at[j0]], gbufA, semA).wait()
            pltpu.sync_copy(gbufA, acc.at[rowsb.at[j0]], add=True)

            @pl.when(j0 + 2 < BCHUNK)
            def _():
                pltpu.async_copy(sp_ref.at[colsb.at[j0 + 2]], gbufA, semA)

            @pl.when(j1 < BCHUNK)
            def _():
                pltpu.make_async_copy(sp_ref.at[colsb.at[j1]], gbufB, semB).wait()
                pltpu.sync_copy(gbufB, acc.at[rowsb.at[j1]], add=True)

            return carry

        lax.fori_loop(0, (BCHUNK + 1) // 2, _step, 0)

    plsc.subcore_barrier()

    base = pl.multiple_of(s * ZROWS, 8)

    @pl.when(s < NS - 1)
    def _():
        pltpu.sync_copy(acc.at[pl.ds(base, ZROWS)],
                        out_ref.at[c, pl.ds(base, ZROWS)])

    @pl.when(s == NS - 1)
    def _():
        pltpu.sync_copy(acc.at[pl.ds(ZROWS * (NS - 1), N - ZROWS * (NS - 1))],
                        out_ref.at[c, pl.ds(ZROWS * (NS - 1),
                                            N - ZROWS * (NS - 1))])


BM = 1000
GRID = N // BM

_row_spec = pl.BlockSpec((BM, D), lambda i: (i, 0))
_col_spec = pl.BlockSpec((BM, 1), lambda i: (i, 0))
_w_spec = pl.BlockSpec((D, D), lambda i: (0, 0))
_b_spec = pl.BlockSpec((1, D), lambda i: (0, 0))
_pair_spec = pl.BlockSpec((2, BM, D), lambda i: (0, i, 0))
_dpair_spec = pl.BlockSpec((2, BM, 1), lambda i: (0, i, 0))


def _k0_body(x_ref, dg_ref, wi_ref, bi_ref, wc_ref, h_ref, sp_ref, dis_ref):
    deg = dg_ref[0] + dg_ref[1] + 1.0
    dis = lax.rsqrt(deg)
    h = jnp.dot(x_ref[...], wi_ref[...],
                preferred_element_type=jnp.float32) + bi_ref[...]
    h_ref[...] = h
    dis_ref[...] = dis
    sp_ref[...] = dis * jnp.dot(h, wc_ref[...],
                                preferred_element_type=jnp.float32)


_k0 = pl.pallas_call(
    _k0_body,
    grid=(GRID,),
    in_specs=[_row_spec, _dpair_spec, _w_spec, _b_spec, _w_spec],
    out_specs=[_row_spec, _row_spec, _col_spec],
    out_shape=[
        jax.ShapeDtypeStruct((N, D), jnp.float32),
        jax.ShapeDtypeStruct((N, D), jnp.float32),
        jax.ShapeDtypeStruct((N, 1), jnp.float32),
    ],
)


def _k1_body(p_ref, sp_ref, h_ref, dis_ref, wl_ref, bl_ref, bc_ref, wc1_ref,
             h1_ref, s1p_ref):
    dis = dis_ref[...]
    t = (dis * (p_ref[0] + p_ref[1] + sp_ref[...]) + bc_ref[...]
         + jnp.dot(h_ref[...], wl_ref[...], preferred_element_type=jnp.float32)
         + bl_ref[...])
    h1 = jnp.maximum(t, 0.0) + h_ref[...]
    h1_ref[...] = h1
    s1p_ref[...] = dis * jnp.dot(h1, wc1_ref[...],
                                 preferred_element_type=jnp.float32)


_k1 = pl.pallas_call(
    _k1_body,
    grid=(GRID,),
    in_specs=[_pair_spec, _row_spec, _row_spec, _col_spec, _w_spec, _b_spec,
              _b_spec, _w_spec],
    out_specs=[_row_spec, _row_spec],
    out_shape=[
        jax.ShapeDtypeStruct((N, D), jnp.float32),
        jax.ShapeDtypeStruct((N, D), jnp.float32),
    ],
)


def _k2_body(p_ref, sp_ref, h_ref, dis_ref, wl_ref, bl_ref, bc_ref, wf_ref,
             bf_ref, out_ref):
    dis = dis_ref[...]
    t = (dis * (p_ref[0] + p_ref[1] + sp_ref[...]) + bc_ref[...]
         + jnp.dot(h_ref[...], wl_ref[...], preferred_element_type=jnp.float32)
         + bl_ref[...])
    h2 = jnp.maximum(t, 0.0) + h_ref[...]
    out_ref[...] = jnp.dot(h2, wf_ref[...],
                           preferred_element_type=jnp.float32) + bf_ref[...]


_k2 = pl.pallas_call(
    _k2_body,
    grid=(GRID,),
    in_specs=[_pair_spec, _row_spec, _row_spec, _col_spec, _w_spec, _b_spec,
              _b_spec, _w_spec, _b_spec],
    out_specs=pl.BlockSpec((BM, D), lambda i: (i, 0)),
    out_shape=jax.ShapeDtypeStruct((N, D), jnp.float32),
)


def _spmm_jnp(sp, edge_index):
    row, col = edge_index[0], edge_index[1]
    keep = row != col
    acc = jnp.zeros_like(sp).at[row].add(
        jnp.where(keep[:, None], sp[col], 0.0))
    return jnp.stack([acc, jnp.zeros_like(acc)])


def kernel(x, edge_index, Winit, binit, Wc0, bc0, Wl0, bl0, Wc1, bc1, Wl1,
           bl1, Wf, bf):
    ei = edge_index.reshape(2, NW, NCHUNK, CHUNK)

    eib = edge_index.reshape(2, NW, NBLK, BCHUNK, CHUNK)

    degp = _deg_kernel(ei)
    deg3 = degp.reshape(NC, NPAD, 1)[:, :N]

    h, s0p, dis = _k0(x, deg3, Winit, binit.reshape(1, D), Wc0)
    parts0 = _spmm_kernel(s0p, eib)
    h1, s1p = _k1(parts0, s0p, h, dis, Wl0, bl0.reshape(1, D),
                  bc0.reshape(1, D), Wc1)
    parts1 = _spmm_kernel(s1p, eib)
    out = _k2(parts1, s1p, h1, dis, Wl1, bl1.reshape(1, D),
              bc1.reshape(1, D), Wf, bf.reshape(1, D))
    return out
